# baseline (speedup 1.0000x reference)
import jax
import jax.numpy as jnp
from jax.experimental import pallas as pl
from jax.experimental.pallas import tpu as pltpu

N_TASKS_K = 8
LATENT_K = 1024
OUT_K = 128
CH = 1024
NC = 8192 // CH
QTR = CH // 4


SLOTS = 2


def _body(task_hbm, x_hbm, w_hbm, bias_hbm, out_hbm,
          wbuf, biasbuf, taskbuf, xbuf, obuf,
          sem_w, sem_bias, sem_task, sems_x, sems_o):
    def x_copy(c, h):
        slot = c % SLOTS
        return pltpu.make_async_copy(
            x_hbm.at[pl.ds(c * CH + h * QTR, QTR), :],
            xbuf.at[slot, pl.ds(h * QTR, QTR), :],
            sems_x.at[slot, h])

    def o_copy(c, h):
        slot = c % SLOTS
        return pltpu.make_async_copy(
            obuf.at[slot, pl.ds(h * 2, 2), :, :],
            out_hbm.at[pl.ds(h * 2, 2), pl.ds(c * CH, CH), :],
            sems_o.at[slot, h])

    cw = pltpu.make_async_copy(w_hbm, wbuf, sem_w)
    cb = pltpu.make_async_copy(bias_hbm, biasbuf, sem_bias)
    ct = pltpu.make_async_copy(task_hbm, taskbuf, sem_task)
    cw.start(); cb.start(); ct.start()
    for c in range(2):
        for h in range(4):
            x_copy(c, h).start()
    cw.wait(); cb.wait(); ct.wait()
    wb = [wbuf[t].astype(jnp.bfloat16) for t in range(N_TASKS_K)]

    for c in range(NC):
        slot = c % SLOTS
        for h in range(4):
            x_copy(c, h).wait()
        if c >= 2:
            for h in range(4):
                o_copy(c - 2, h).wait()
        xb = xbuf[slot].astype(jnp.bfloat16)
        tb = taskbuf[0, pl.ds(c * CH, CH)]
        for t in range(N_TASKS_K):
            yt = jax.lax.dot_general(
                xb, wb[t], (((1,), (1,)), ((), ())),
                preferred_element_type=jnp.float32)
            yt = yt + biasbuf[0, pl.ds(t * OUT_K, OUT_K)][None, :]
            m = (tb == t).astype(jnp.float32)[:, None]
            obuf[slot, t, :, :] = yt * m
        for h in range(4):
            o_copy(c, h).start()
        if c + 2 < NC:
            for h in range(4):
                x_copy(c + 2, h).start()
    for h in range(4):
        o_copy(NC - 2, h).wait()
    for h in range(4):
        o_copy(NC - 1, h).wait()


def kernel(output_latents, output_task_index, W, b):
    Bsz, T, D = output_latents.shape
    N = Bsz * T
    x = output_latents.reshape(N, D)
    task = output_task_index.reshape(1, N).astype(jnp.int32)
    bias_row = b.reshape(1, N_TASKS_K * OUT_K)

    out = pl.pallas_call(
        _body,
        in_specs=[pl.BlockSpec(memory_space=pl.ANY)] * 4,
        out_specs=pl.BlockSpec(memory_space=pl.ANY),
        out_shape=jax.ShapeDtypeStruct((N_TASKS_K, N, OUT_K), jnp.float32),
        scratch_shapes=[
            pltpu.VMEM((N_TASKS_K, OUT_K, LATENT_K), jnp.float32),
            pltpu.VMEM((1, N_TASKS_K * OUT_K), jnp.float32),
            pltpu.VMEM((1, N), jnp.int32),
            pltpu.VMEM((SLOTS, CH, D), jnp.float32),
            pltpu.VMEM((SLOTS, N_TASKS_K, CH, OUT_K), jnp.float32),
            pltpu.SemaphoreType.DMA,
            pltpu.SemaphoreType.DMA,
            pltpu.SemaphoreType.DMA,
            pltpu.SemaphoreType.DMA((SLOTS, 4)),
            pltpu.SemaphoreType.DMA((SLOTS, 4)),
        ],
    )(task, x, W, bias_row)
    return out.reshape(N_TASKS_K, Bsz, T, OUT_K)


# speedup vs baseline: 1.5572x; 1.5572x over previous
import jax
import jax.numpy as jnp
from jax.experimental import pallas as pl
from jax.experimental.pallas import tpu as pltpu

N_TASKS_K = 8
LATENT_K = 1024
OUT_K = 128
CH = 1024
NC = 8192 // CH
QTR = CH // 4


SLOTS = 2


def _body(task_hbm, x_hbm, w_hbm, bias_hbm, out_hbm,
          wbuf, biasbuf, taskbuf, xbuf, obuf,
          sem_w, sem_bias, sem_task, sems_x, sems_o):
    def x_copy(c, h):
        slot = c % SLOTS
        return pltpu.make_async_copy(
            x_hbm.at[pl.ds(c * CH + h * QTR, QTR), :],
            xbuf.at[slot, pl.ds(h * QTR, QTR), :],
            sems_x.at[slot, h])

    def o_copy(c, h):
        slot = c % SLOTS
        return pltpu.make_async_copy(
            obuf.at[slot, pl.ds(h * 2, 2), :, :],
            out_hbm.at[pl.ds(h * 2, 2), pl.ds(c * CH, CH), :],
            sems_o.at[slot, h])

    cw = pltpu.make_async_copy(w_hbm, wbuf, sem_w)
    cb = pltpu.make_async_copy(bias_hbm, biasbuf, sem_bias)
    ct = pltpu.make_async_copy(task_hbm, taskbuf, sem_task)
    cw.start(); cb.start(); ct.start()
    for c in range(2):
        for h in range(4):
            x_copy(c, h).start()
    cw.wait(); cb.wait(); ct.wait()
    w2d = wbuf[...].reshape(N_TASKS_K * OUT_K, LATENT_K)
    wt = jnp.transpose(w2d).astype(jnp.bfloat16)

    for c in range(NC):
        slot = c % SLOTS
        for h in range(4):
            x_copy(c, h).wait()
        if c >= 2:
            for h in range(4):
                o_copy(c - 2, h).wait()
        xb = xbuf[slot].astype(jnp.bfloat16)
        y = jnp.dot(xb, wt, preferred_element_type=jnp.float32)
        y = y + biasbuf[...]
        tb = taskbuf[0, pl.ds(c * CH, CH)]
        for t in range(N_TASKS_K):
            m = (tb == t).astype(jnp.float32)[:, None]
            obuf[slot, t, :, :] = y[:, t * OUT_K:(t + 1) * OUT_K] * m
        for h in range(4):
            o_copy(c, h).start()
        if c + 2 < NC:
            for h in range(4):
                x_copy(c + 2, h).start()
    for h in range(4):
        o_copy(NC - 2, h).wait()
    for h in range(4):
        o_copy(NC - 1, h).wait()


def kernel(output_latents, output_task_index, W, b):
    Bsz, T, D = output_latents.shape
    N = Bsz * T
    x = output_latents.reshape(N, D)
    task = output_task_index.reshape(1, N).astype(jnp.int32)
    bias_row = b.reshape(1, N_TASKS_K * OUT_K)

    out = pl.pallas_call(
        _body,
        in_specs=[pl.BlockSpec(memory_space=pl.ANY)] * 4,
        out_specs=pl.BlockSpec(memory_space=pl.ANY),
        out_shape=jax.ShapeDtypeStruct((N_TASKS_K, N, OUT_K), jnp.float32),
        scratch_shapes=[
            pltpu.VMEM((N_TASKS_K, OUT_K, LATENT_K), jnp.float32),
            pltpu.VMEM((1, N_TASKS_K * OUT_K), jnp.float32),
            pltpu.VMEM((1, N), jnp.int32),
            pltpu.VMEM((SLOTS, CH, D), jnp.float32),
            pltpu.VMEM((SLOTS, N_TASKS_K, CH, OUT_K), jnp.float32),
            pltpu.SemaphoreType.DMA,
            pltpu.SemaphoreType.DMA,
            pltpu.SemaphoreType.DMA,
            pltpu.SemaphoreType.DMA((SLOTS, 4)),
            pltpu.SemaphoreType.DMA((SLOTS, 4)),
        ],
    )(task, x, W, bias_row)
    return out.reshape(N_TASKS_K, Bsz, T, OUT_K)
